# baseline (speedup 1.0000x reference)
import functools

import jax
import jax.numpy as jnp
from jax import lax
from jax.experimental import pallas as pl
from jax.experimental.pallas import tpu as pltpu
from jax.experimental.pallas import tpu_sc as plsc

_LATENTS = (32, 64, 96, 128, 192, 256)
_NH = 6
_DIN = 73
_DP = 128
_DMAX = 256
_TILE = 512
_NT = 40
_P = _NT * _TILE
_BATCH = 16384
_NW = 32
_PB = _P // _NW
_NCH = _PB // 128


def _gather_body(src_hbm, idx_hbm, out_hbm, idx_v, buf_v, sem):
    wid = lax.axis_index("s") * 2 + lax.axis_index("c")
    base = wid * _PB
    for j in range(_NCH):
        pltpu.sync_copy(idx_hbm.at[pl.ds(base + j * 128, 128)], idx_v.at[j])
    cps = [pltpu.async_copy(src_hbm.at[idx_v.at[j]],
                            buf_v.at[pl.ds(j * 128, 128)], sem)
           for j in range(_NCH)]
    for cp in cps:
        cp.wait()
    pltpu.sync_copy(buf_v, out_hbm.at[pl.ds(base, _PB)])


def _sc_gather(src, idx2d):
    mesh = plsc.VectorSubcoreMesh(core_axis_name="c", subcore_axis_name="s", num_cores=2, num_subcores=16)
    k = functools.partial(
        pl.kernel,
        out_type=jax.ShapeDtypeStruct((_P, _DP), jnp.float32),
        mesh=mesh,
        scratch_types=[
            pltpu.VMEM((_NCH, 128), jnp.int32),
            pltpu.VMEM((_PB, _DP), jnp.float32),
            pltpu.SemaphoreType.DMA,
        ],
    )(_gather_body)
    return k(src, idx2d)


def _scatter_body(src_hbm, idx_hbm, out_hbm, idx_v, buf_v, sem):
    wid = lax.axis_index("s") * 2 + lax.axis_index("c")
    base = wid * _PB
    for j in range(_NCH):
        pltpu.sync_copy(idx_hbm.at[pl.ds(base + j * 128, 128)], idx_v.at[j])
    prev = None
    for j in range(_NCH):
        b = j % 2
        pltpu.sync_copy(src_hbm.at[pl.ds(base + j * 128, 128)], buf_v.at[b])
        cp = pltpu.async_copy(buf_v.at[b], out_hbm.at[idx_v.at[j]], sem)
        if prev is not None:
            prev.wait()
        prev = cp
    prev.wait()


def _sc_scatter(src, idx2d):
    mesh = plsc.VectorSubcoreMesh(core_axis_name="c", subcore_axis_name="s", num_cores=2, num_subcores=16)
    k = functools.partial(
        pl.kernel,
        out_type=jax.ShapeDtypeStruct((_BATCH + 1, _DMAX), jnp.float32),
        mesh=mesh,
        scratch_types=[
            pltpu.VMEM((_NCH, 128), jnp.int32),
            pltpu.VMEM((2, 128, _DMAX), jnp.float32),
            pltpu.SemaphoreType.DMA,
        ],
    )(_scatter_body)
    return k(src, idx2d)


def _mlp_body(te_ref, dt_ref, x_ref, a_ref, w1_ref, b1_ref, w2_ref, b2_ref,
              w3_ref, b3_ref, lnw_ref, lnb_ref, o_ref):
    i = pl.program_id(0)
    e = te_ref[i]
    d = dt_ref[i]
    x = x_ref[...]
    h = jax.lax.dot_general(x, w1_ref[0], (((1,), (1,)), ((), ())),
                            preferred_element_type=jnp.float32)
    h = h + b1_ref[0]
    h = jnp.where(h >= 0, h, a_ref[e, 0] * h)
    h = jax.lax.dot_general(h, w2_ref[0], (((1,), (1,)), ((), ())),
                            preferred_element_type=jnp.float32)
    h = h + b2_ref[0]
    h = jnp.where(h >= 0, h, a_ref[e, 1] * h)
    h = jax.lax.dot_general(h, w3_ref[0], (((1,), (1,)), ((), ())),
                            preferred_element_type=jnp.float32)
    h = h + b3_ref[0]
    inv_d = 1.0 / d.astype(jnp.float32)
    mu = jnp.sum(h, axis=1, keepdims=True) * inv_d
    col = jax.lax.broadcasted_iota(jnp.int32, h.shape, 1)
    diff = jnp.where(col < d, h - mu, 0.0)
    var = jnp.sum(diff * diff, axis=1, keepdims=True) * inv_d
    o_ref[...] = diff * jax.lax.rsqrt(var + 1e-5) * lnw_ref[0] + lnb_ref[0]


def _routed_mlp(tile_expert, d_tile, gathered, a_all, w1s, b1s, w2s, b2s,
                w3s, b3s, lnws, lnbs):
    wspec = lambda *shp: pl.BlockSpec((1,) + shp, lambda i, te, dt:
                                      (te[i],) + (0,) * len(shp))
    grid_spec = pltpu.PrefetchScalarGridSpec(
        num_scalar_prefetch=2,
        grid=(_NT,),
        in_specs=[
            pl.BlockSpec((_TILE, _DP), lambda i, te, dt: (i, 0)),
            pl.BlockSpec(memory_space=pltpu.SMEM),
            wspec(_DP, _DP),
            wspec(1, _DP),
            wspec(64, _DP),
            wspec(1, 64),
            wspec(_DMAX, 64),
            wspec(1, _DMAX),
            wspec(1, _DMAX),
            wspec(1, _DMAX),
        ],
        out_specs=pl.BlockSpec((_TILE, _DMAX), lambda i, te, dt: (i, 0)),
    )
    return pl.pallas_call(
        _mlp_body,
        grid_spec=grid_spec,
        out_shape=jax.ShapeDtypeStruct((_P, _DMAX), jnp.float32),
    )(tile_expert, d_tile, gathered, a_all, w1s, b1s, w2s, b2s, w3s, b3s,
      lnws, lnbs)


def kernel(equalized_symbol, csi_context, noise_power, rate_one_hot, params):
    b = equalized_symbol.shape[0]
    combined = jnp.concatenate(
        [equalized_symbol, csi_context, noise_power[:, None],
         jnp.zeros((b, _DP - _DIN), jnp.float32)], axis=1)
    combined = jnp.concatenate(
        [combined, jnp.zeros((1, _DP), jnp.float32)], axis=0)

    e = jnp.argmax(rate_one_hot, axis=1).astype(jnp.int32)
    oh = (e[:, None] == jnp.arange(_NH, dtype=jnp.int32)[None, :]).astype(jnp.int32)
    cum = jnp.cumsum(oh, axis=0)
    rank = jnp.take_along_axis(cum, e[:, None], axis=1)[:, 0] - 1
    counts = cum[-1]
    seg = ((counts + _TILE - 1) // _TILE) * _TILE
    bounds = jnp.cumsum(seg)
    starts = bounds - seg
    slot = starts[e] + rank
    tfs = jnp.full((_P,), _BATCH, jnp.int32).at[slot].set(
        jnp.arange(_BATCH, dtype=jnp.int32))
    tile_pos = jnp.arange(_NT, dtype=jnp.int32) * _TILE
    tile_expert = jnp.sum((tile_pos[:, None] >= bounds[None, :]).astype(jnp.int32),
                          axis=1)
    tile_expert = jnp.where(tile_expert >= _NH, 0, tile_expert)
    d_tile = jnp.array(_LATENTS, jnp.int32)[tile_expert]

    w1s = jnp.stack([jnp.pad(p['W1'], ((0, 0), (0, _DP - _DIN))) for p in params])
    b1s = jnp.stack([p['b1'][None, :] for p in params])
    w2s = jnp.stack([p['W2'] for p in params])
    b2s = jnp.stack([p['b2'][None, :] for p in params])
    w3s = jnp.stack([jnp.pad(p['W3'], ((0, _DMAX - p['W3'].shape[0]), (0, 0)))
                     for p in params])
    b3s = jnp.stack([jnp.pad(p['b3'], (0, _DMAX - p['b3'].shape[0]))[None, :]
                     for p in params])
    lnws = jnp.stack([jnp.pad(p['ln_w'], (0, _DMAX - p['ln_w'].shape[0]))[None, :]
                      for p in params])
    lnbs = jnp.stack([jnp.pad(p['ln_b'], (0, _DMAX - p['ln_b'].shape[0]))[None, :]
                      for p in params])
    a_all = jnp.stack([jnp.concatenate([p['a1'], p['a2']]) for p in params])

    gathered = _sc_gather(combined, tfs)
    out_sorted = _routed_mlp(tile_expert, d_tile, gathered, a_all, w1s, b1s,
                             w2s, b2s, w3s, b3s, lnws, lnbs)
    out_pad = _sc_scatter(out_sorted, tfs)
    return out_pad[:_BATCH]


# speedup vs baseline: 3.3094x; 3.3094x over previous
import jax
import jax.numpy as jnp
from jax.experimental import pallas as pl
from jax.experimental.pallas import tpu as pltpu

_LATENTS = (32, 64, 96, 128, 192, 256)
_NH = 6
_DIN = 73
_DP = 128
_DMAX = 256
_TILE = 512


def _fused_body(x_ref, r_ref, w1_ref, b1_ref, a_ref, w2_ref, b2_ref,
                w3_ref, b3_ref, lnw_ref, lnb_ref, o_ref):
    x = x_ref[...]
    best = r_ref[:, 0:1]
    e = jnp.zeros((x.shape[0], 1), jnp.int32)
    for j in range(1, _NH):
        rj = r_ref[:, j:j + 1]
        m = rj > best
        e = jnp.where(m, j, e)
        best = jnp.maximum(best, rj)

    h1 = jax.lax.dot_general(x, w1_ref[...], (((1,), (0,)), ((), ())),
                             preferred_element_type=jnp.float32)
    h1 = h1 + b1_ref[...]

    acc = jnp.zeros((x.shape[0], _DMAX), jnp.float32)
    for i in range(_NH):
        d = _LATENTS[i]
        h = h1[:, i * _DP:(i + 1) * _DP]
        h = jnp.where(h >= 0, h, a_ref[i, 0] * h).astype(jnp.bfloat16)
        h = jax.lax.dot_general(h, w2_ref[i], (((1,), (1,)), ((), ())),
                                preferred_element_type=jnp.float32)
        h = h + b2_ref[i]
        h = jnp.where(h >= 0, h, a_ref[i, 1] * h).astype(jnp.bfloat16)
        h = jax.lax.dot_general(h, w3_ref[i], (((1,), (1,)), ((), ())),
                                preferred_element_type=jnp.float32)
        h = h + b3_ref[i]
        mu = jnp.sum(h, axis=1, keepdims=True) * (1.0 / d)
        col = jax.lax.broadcasted_iota(jnp.int32, h.shape, 1)
        diff = jnp.where(col < d, h - mu, 0.0)
        var = jnp.sum(diff * diff, axis=1, keepdims=True) * (1.0 / d)
        y = diff * jax.lax.rsqrt(var + 1e-5) * lnw_ref[i] + lnb_ref[i]
        mask = (e == i).astype(jnp.float32)
        acc = acc + mask * y
    o_ref[...] = acc


def kernel(equalized_symbol, csi_context, noise_power, rate_one_hot, params):
    b = equalized_symbol.shape[0]
    combined = jnp.concatenate(
        [equalized_symbol, csi_context, noise_power[:, None],
         jnp.zeros((b, _DP - _DIN), jnp.float32)], axis=1).astype(jnp.bfloat16)

    w1s = jnp.concatenate(
        [jnp.pad(p['W1'], ((0, 0), (0, _DP - _DIN))).T for p in params],
        axis=1).astype(jnp.bfloat16)
    b1s = jnp.concatenate([p['b1'] for p in params])[None, :]
    w2s = jnp.stack([p['W2'] for p in params]).astype(jnp.bfloat16)
    b2s = jnp.stack([p['b2'][None, :] for p in params])
    w3s = jnp.stack([jnp.pad(p['W3'], ((0, _DMAX - p['W3'].shape[0]), (0, 0)))
                     for p in params]).astype(jnp.bfloat16)
    b3s = jnp.stack([jnp.pad(p['b3'], (0, _DMAX - p['b3'].shape[0]))[None, :]
                     for p in params])
    lnws = jnp.stack([jnp.pad(p['ln_w'], (0, _DMAX - p['ln_w'].shape[0]))[None, :]
                      for p in params])
    lnbs = jnp.stack([jnp.pad(p['ln_b'], (0, _DMAX - p['ln_b'].shape[0]))[None, :]
                      for p in params])
    a_all = jnp.stack([jnp.concatenate([p['a1'], p['a2']]) for p in params])

    grid = (b // _TILE,)
    full = lambda shp: pl.BlockSpec(shp, lambda i: (0,) * len(shp))
    out = pl.pallas_call(
        _fused_body,
        grid=grid,
        in_specs=[
            pl.BlockSpec((_TILE, _DP), lambda i: (i, 0)),
            pl.BlockSpec((_TILE, _NH), lambda i: (i, 0)),
            full((_DP, _NH * _DP)),
            full((1, _NH * _DP)),
            pl.BlockSpec(memory_space=pltpu.SMEM),
            full((_NH, 64, _DP)),
            full((_NH, 1, 64)),
            full((_NH, _DMAX, 64)),
            full((_NH, 1, _DMAX)),
            full((_NH, 1, _DMAX)),
            full((_NH, 1, _DMAX)),
        ],
        out_specs=pl.BlockSpec((_TILE, _DMAX), lambda i: (i, 0)),
        out_shape=jax.ShapeDtypeStruct((b, _DMAX), jnp.float32),
    )(combined, rate_one_hot, w1s, b1s, a_all, w2s, b2s, w3s, b3s, lnws, lnbs)
    return out


# speedup vs baseline: 3.4635x; 1.0466x over previous
import jax
import jax.numpy as jnp
from jax.experimental import pallas as pl
from jax.experimental.pallas import tpu as pltpu

_LATENTS = (32, 64, 96, 128, 192, 256)
_NH = 6
_DIN = 73
_DP = 128
_DMAX = 256
_TILE = 512


def _dot(a, b):
    return jax.lax.dot_general(a, b, (((1,), (0,)), ((), ())),
                               preferred_element_type=jnp.float32)


def _fused_body(x_ref, r_ref, lt_ref, w1_ref, b1_ref, a1_ref, a_ref, w2_ref,
                b2_ref, w3_ref, bc_ref, lnb_ref, aux_ref, o_ref):
    r = r_ref[...]
    best = jnp.max(r, axis=1, keepdims=True)
    eqm = (r == best).astype(jnp.float32)
    cums = _dot(eqm, lt_ref[...])
    fm = eqm * (cums == 1.0).astype(jnp.float32)

    x = x_ref[...]
    h1 = _dot(x, w1_ref[...]) + b1_ref[...]
    h1 = jnp.where(h1 >= 0, h1, a1_ref[...] * h1)

    acc = jnp.zeros((x.shape[0], 2 * _DMAX), jnp.float32)
    for i in range(_NH):
        h = h1[:, i * _DP:(i + 1) * _DP]
        h2 = jax.lax.dot_general(h, w2_ref[i], (((1,), (1,)), ((), ())),
                                 preferred_element_type=jnp.float32)
        h2 = h2 + b2_ref[i]
        h2 = jnp.where(h2 >= 0, h2, a_ref[i, 1] * h2)
        h2 = h2 * fm[:, i:i + 1]
        acc = acc + _dot(h2, w3_ref[i])
    acc = acc + _dot(fm, bc_ref[...])
    z = acc[:, :_DMAX]
    u = acc[:, _DMAX:]
    mix = _dot(fm, aux_ref[...])
    ss = _dot(u * u, jnp.ones((_DMAX, 1), jnp.float32))
    rs = jax.lax.rsqrt(ss * mix[:, 0:1] + 1e-5)
    lnb = _dot(fm, lnb_ref[...])
    o_ref[...] = z * rs + lnb


def kernel(equalized_symbol, csi_context, noise_power, rate_one_hot, params):
    b = equalized_symbol.shape[0]
    combined = jnp.concatenate(
        [equalized_symbol, csi_context, noise_power[:, None],
         jnp.zeros((b, _DP - _DIN), jnp.float32)], axis=1)

    w1s = jnp.concatenate(
        [jnp.pad(p['W1'], ((0, 0), (0, _DP - _DIN))).T for p in params],
        axis=1)
    b1s = jnp.concatenate([p['b1'] for p in params])[None, :]
    a1rep = jnp.concatenate(
        [jnp.broadcast_to(p['a1'], (_DP,)) for p in params])[None, :]
    w2s = jnp.stack([p['W2'] for p in params])
    b2s = jnp.stack([p['b2'][None, :] for p in params])
    a_all = jnp.stack([jnp.concatenate([p['a1'], p['a2']]) for p in params])

    w3cs, bcs, lnbs = [], [], []
    for i, p in enumerate(params):
        d = _LATENTS[i]
        w3 = p['W3']
        b3 = p['b3']
        wbar = jnp.mean(w3, axis=0, keepdims=True)
        bbar = jnp.mean(b3)
        w3p = jnp.pad(w3 - wbar, ((0, _DMAX - d), (0, 0)))
        b3p = jnp.pad(b3 - bbar, (0, _DMAX - d))
        lnw = jnp.pad(p['ln_w'], (0, _DMAX - d))
        w3sc = lnw[:, None] * w3p
        b3sc = lnw * b3p
        w3cs.append(jnp.concatenate([w3sc.T, w3p.T], axis=1))
        bcs.append(jnp.concatenate([b3sc, b3p])[None, :])
        lnbs.append(jnp.pad(p['ln_b'], (0, _DMAX - d)))
    w3cat = jnp.stack(w3cs)
    bcat = jnp.stack(bcs)
    lnbcat = jnp.stack(lnbs)
    lt = jnp.tril(jnp.ones((_NH, _NH), jnp.float32))
    aux = jnp.stack([1.0 / jnp.array(_LATENTS, jnp.float32),
                     jnp.zeros((_NH,), jnp.float32)], axis=1)

    grid = (b // _TILE,)
    full = lambda shp: pl.BlockSpec(shp, lambda i: (0,) * len(shp))
    out = pl.pallas_call(
        _fused_body,
        grid=grid,
        in_specs=[
            pl.BlockSpec((_TILE, _DP), lambda i: (i, 0)),
            pl.BlockSpec((_TILE, _NH), lambda i: (i, 0)),
            full((_NH, _NH)),
            full((_DP, _NH * _DP)),
            full((1, _NH * _DP)),
            full((1, _NH * _DP)),
            pl.BlockSpec(memory_space=pltpu.SMEM),
            full((_NH, 64, _DP)),
            full((_NH, 1, 64)),
            full((_NH, 64, 2 * _DMAX)),
            full((_NH, 2 * _DMAX)),
            full((_NH, _DMAX)),
            full((_NH, 2)),
        ],
        out_specs=pl.BlockSpec((_TILE, _DMAX), lambda i: (i, 0)),
        out_shape=jax.ShapeDtypeStruct((b, _DMAX), jnp.float32),
    )(combined, rate_one_hot, lt, w1s, b1s, a1rep, a_all, w2s, b2s,
      w3cat, bcat.reshape(_NH, 2 * _DMAX), lnbcat, aux)
    return out


# speedup vs baseline: 3.6717x; 1.0601x over previous
import jax
import jax.numpy as jnp
from jax.experimental import pallas as pl
from jax.experimental.pallas import tpu as pltpu

_LATENTS = (32, 64, 96, 128, 192, 256)
_NH = 6
_DIN = 73
_DMAX = 256
_TILE = 512


def _dot(a, b):
    return jax.lax.dot_general(a, b, (((1,), (1,)), ((), ())),
                               preferred_element_type=jnp.float32)


def _fused_body(eq_ref, csi_ref, np_ref, r_ref, *refs):
    o_ref = refs[-1]
    pr = refs[:-1]

    r = r_ref[...]
    best = jnp.max(r, axis=1, keepdims=True)
    eqm = (r == best).astype(jnp.float32)
    lt = jnp.tril(jnp.ones((_NH, _NH), jnp.float32))
    cums = jax.lax.dot_general(eqm, lt, (((1,), (0,)), ((), ())),
                               preferred_element_type=jnp.float32)
    fm = eqm * (cums == 1.0).astype(jnp.float32)

    x = jnp.concatenate([eq_ref[...], csi_ref[...], np_ref[...]], axis=1)

    T = x.shape[0]
    zsum = jnp.zeros((T, _DMAX), jnp.float32)
    lnbsel = jnp.zeros((T, _DMAX), jnp.float32)
    ss = jnp.zeros((T, 1), jnp.float32)
    invd = jnp.zeros((T, 1), jnp.float32)
    for i in range(_NH):
        (w1_r, b1_r, a1_r, w2_r, b2_r, a2_r, w3_r, b3_r, lnw_r, lnb_r) = \
            pr[10 * i:10 * (i + 1)]
        d = _LATENTS[i]
        fmi = fm[:, i:i + 1]
        h = _dot(x, w1_r[...]) + b1_r[...]
        h = jnp.where(h >= 0, h, a1_r[0] * h)
        h = _dot(h, w2_r[...]) + b2_r[...]
        h = jnp.where(h >= 0, h, a2_r[0] * h)
        h = h * fmi

        w3 = w3_r[...]
        wbar = jnp.mean(w3, axis=0, keepdims=True)
        w3p = w3 - wbar
        lnw = lnw_r[...]
        b3 = b3_r[...]
        b3p = b3 - jnp.mean(b3)
        z = _dot(h, w3p * lnw[:, None]) + fmi * (lnw * b3p)
        u = _dot(h, w3p) + fmi * b3p
        pad = jnp.zeros((T, _DMAX - d), jnp.float32) if d < _DMAX else None
        if pad is not None:
            zsum = zsum + jnp.concatenate([z, pad], axis=1)
            lnbsel = lnbsel + jnp.concatenate([fmi * lnb_r[...], pad], axis=1)
        else:
            zsum = zsum + z
            lnbsel = lnbsel + fmi * lnb_r[...]
        ss = ss + _dot(u * u, jnp.ones((1, d), jnp.float32))
        invd = invd + fmi * (1.0 / d)

    rs = jax.lax.rsqrt(ss * invd + 1e-5)
    o_ref[...] = zsum * rs + lnbsel


def kernel(equalized_symbol, csi_context, noise_power, rate_one_hot, params):
    b = equalized_symbol.shape[0]
    grid = (b // _TILE,)
    full = lambda a: pl.BlockSpec(a.shape, lambda i: (0,) * a.ndim)
    smem = pl.BlockSpec(memory_space=pltpu.SMEM)

    pargs, pspecs = [], []
    for p in params:
        for k in ('W1', 'b1', 'a1', 'W2', 'b2', 'a2', 'W3', 'b3', 'ln_w', 'ln_b'):
            v = p[k]
            pargs.append(v)
            pspecs.append(smem if k in ('a1', 'a2') else full(v))

    out = pl.pallas_call(
        _fused_body,
        grid=grid,
        in_specs=[
            pl.BlockSpec((_TILE, 8), lambda i: (i, 0)),
            pl.BlockSpec((_TILE, 64), lambda i: (i, 0)),
            pl.BlockSpec((_TILE, 1), lambda i: (i, 0)),
            pl.BlockSpec((_TILE, _NH), lambda i: (i, 0)),
            *pspecs,
        ],
        out_specs=pl.BlockSpec((_TILE, _DMAX), lambda i: (i, 0)),
        out_shape=jax.ShapeDtypeStruct((b, _DMAX), jnp.float32),
    )(equalized_symbol, csi_context, noise_power[:, None], rate_one_hot,
      *pargs)
    return out
